# baseline (speedup 1.0000x reference)
import jax
import jax.numpy as jnp
from jax import lax
from jax.experimental import pallas as pl
from jax.experimental.pallas import tpu as pltpu
from jax.experimental.pallas import tpu_sc as plsc

N_CLS = 100000
N_CO_CLS = 20
N_CO_GRAPH = 9
CTX_DIM = 512
B = 1024
NC, NS = 2, 16
NW = NC * NS
BPW = B // NW
CHUNK = 8
NCH = BPW // CHUNK
ALLR = 2


def _sc_body(ci_hbm, gi_hbm, hi_hbm, sgl_hbm, all_hbm, cls_hbm, gph_hbm,
             out_hbm, all_sh, clstab_sh, all_v, ci_v, gi_v, hi_v, gphtab_v,
             sgl_v,
             sem_g, sem_o, sem_a):
    sid = lax.axis_index("s")
    wid = sid * NC + lax.axis_index("c")
    base = wid * BPW

    @pl.when(sid == 0)
    def _stage_shared():
        for e in range(ALLR):
            pltpu.sync_copy(all_hbm.at[0], all_sh.at[e])
        pltpu.sync_copy(cls_hbm, clstab_sh)
    for e in range(ALLR):
        pltpu.sync_copy(all_hbm.at[0], all_v.at[e])
    pltpu.sync_copy(gph_hbm, gphtab_v)
    pltpu.sync_copy(ci_hbm.at[pl.ds(base, BPW)], ci_v)
    pltpu.sync_copy(gi_hbm.at[pl.ds(base, BPW)], gi_v)
    pltpu.sync_copy(hi_hbm.at[pl.ds(base, BPW)], hi_v)
    plsc.subcore_barrier()

    lanes = lax.iota(jnp.int32, 16)

    def scalar_at(vref, k):
        vec = vref[pl.ds((k // 16) * 16, 16)]
        return jnp.sum(jnp.where(lanes == (k % 16), vec, 0))

    for j in range(NCH):
        sl = pl.ds(j * CHUNK, CHUNK)
        r0 = base + j * CHUNK
        g1 = pltpu.async_copy(sgl_hbm.at[ci_v.at[sl]], sgl_v, sem_g)
        outs = [pltpu.async_copy(
            all_sh if k % 2 == 0 else all_v,
            out_hbm.at[pl.ds(r0 + k * ALLR, ALLR), pl.ds(0, 16), :],
            sem_a) for k in range(CHUNK // ALLR)]
        for e in range(CHUNK):
            c = scalar_at(gi_v, j * CHUNK + e)
            outs.append(pltpu.async_copy(
                clstab_sh.at[c],
                out_hbm.at[r0 + e, pl.ds(16, 8), :], sem_o))
            g = scalar_at(hi_v, j * CHUNK + e)
            outs.append(pltpu.async_copy(
                gphtab_v.at[g],
                out_hbm.at[r0 + e, pl.ds(24, 4), :], sem_a))
        g1.wait()
        outs.append(pltpu.async_copy(
            sgl_v, out_hbm.at[pl.ds(r0, CHUNK), pl.ds(28, 4), :], sem_o))
        for d in outs:
            d.wait()


def kernel(cls_idx, cls_group_idx, graph_group_idx, ctx_single, ctx_all,
           ctx_cls, ctx_graph):
    mesh = plsc.VectorSubcoreMesh(core_axis_name="c", subcore_axis_name="s",
                                  num_cores=NC, num_subcores=NS)
    run = pl.kernel(
        _sc_body,
        out_type=jax.ShapeDtypeStruct((B, 32, CTX_DIM), jnp.float32),
        mesh=mesh,
        compiler_params=pltpu.CompilerParams(use_tc_tiling_on_sc=True,
                                             needs_layout_passes=False),
        scratch_types=[
            pltpu.VMEM_SHARED((ALLR, 16, CTX_DIM), jnp.float32),
            pltpu.VMEM_SHARED((N_CO_CLS, 8, CTX_DIM), jnp.float32),
            pltpu.VMEM((ALLR, 16, CTX_DIM), jnp.float32),
            pltpu.VMEM((BPW,), jnp.int32),
            pltpu.VMEM((BPW,), jnp.int32),
            pltpu.VMEM((BPW,), jnp.int32),
            pltpu.VMEM((N_CO_GRAPH, 4, CTX_DIM), jnp.float32),
            pltpu.VMEM((CHUNK, 4, CTX_DIM), jnp.float32),
            pltpu.SemaphoreType.DMA,
            pltpu.SemaphoreType.DMA,
            pltpu.SemaphoreType.DMA,
        ],
    )
    return run(cls_idx, cls_group_idx, graph_group_idx, ctx_single, ctx_all,
               ctx_cls, ctx_graph)


# speedup vs baseline: 1.0042x; 1.0042x over previous
import jax
import jax.numpy as jnp
from jax import lax
from jax.experimental import pallas as pl
from jax.experimental.pallas import tpu as pltpu
from jax.experimental.pallas import tpu_sc as plsc

N_CLS = 100000
N_CO_CLS = 20
N_CO_GRAPH = 9
CTX_DIM = 512
B = 1024
NC, NS = 2, 16
NW = NC * NS
BPW = B // NW
CHUNK = 8
NCH = BPW // CHUNK
ALLR = 2


def _sc_body(ci_hbm, gi_hbm, hi_hbm, sgl_hbm, all_hbm, cls_hbm, gph_hbm,
             out_hbm, all_sh, clstab_sh, ci_v, gi_v, hi_v, gphtab_v, sgl_v,
             sem_g, sem_o, sem_a):
    sid = lax.axis_index("s")
    wid = sid * NC + lax.axis_index("c")
    base = wid * BPW

    @pl.when(sid == 0)
    def _stage_shared():
        for e in range(ALLR):
            pltpu.sync_copy(all_hbm.at[0], all_sh.at[e])
        pltpu.sync_copy(cls_hbm, clstab_sh)
    pltpu.sync_copy(gph_hbm, gphtab_v)
    pltpu.sync_copy(ci_hbm.at[pl.ds(base, BPW)], ci_v)
    pltpu.sync_copy(gi_hbm.at[pl.ds(base, BPW)], gi_v)
    pltpu.sync_copy(hi_hbm.at[pl.ds(base, BPW)], hi_v)
    plsc.subcore_barrier()

    lanes = lax.iota(jnp.int32, 16)

    def scalar_at(vref, k):
        vec = vref[pl.ds((k // 16) * 16, 16)]
        return jnp.sum(jnp.where(lanes == (k % 16), vec, 0))

    for j in range(NCH):
        sl = pl.ds(j * CHUNK, CHUNK)
        r0 = base + j * CHUNK
        g1 = pltpu.async_copy(sgl_hbm.at[ci_v.at[sl]], sgl_v, sem_g)
        outs = [pltpu.async_copy(
            all_sh, out_hbm.at[pl.ds(r0 + k * ALLR, ALLR), pl.ds(0, 16), :],
            sem_a) for k in range(CHUNK // ALLR)]
        for e in range(CHUNK):
            c = scalar_at(gi_v, j * CHUNK + e)
            outs.append(pltpu.async_copy(
                clstab_sh.at[c],
                out_hbm.at[r0 + e, pl.ds(16, 8), :], sem_o))
            g = scalar_at(hi_v, j * CHUNK + e)
            outs.append(pltpu.async_copy(
                gphtab_v.at[g],
                out_hbm.at[r0 + e, pl.ds(24, 4), :], sem_a))
        g1.wait()
        outs.append(pltpu.async_copy(
            sgl_v, out_hbm.at[pl.ds(r0, CHUNK), pl.ds(28, 4), :], sem_o))
        for d in outs:
            d.wait()


def kernel(cls_idx, cls_group_idx, graph_group_idx, ctx_single, ctx_all,
           ctx_cls, ctx_graph):
    mesh = plsc.VectorSubcoreMesh(core_axis_name="c", subcore_axis_name="s",
                                  num_cores=NC, num_subcores=NS)
    run = pl.kernel(
        _sc_body,
        out_type=jax.ShapeDtypeStruct((B, 32, CTX_DIM), jnp.float32),
        mesh=mesh,
        compiler_params=pltpu.CompilerParams(use_tc_tiling_on_sc=True,
                                             needs_layout_passes=False),
        scratch_types=[
            pltpu.VMEM_SHARED((ALLR, 16, CTX_DIM), jnp.float32),
            pltpu.VMEM_SHARED((N_CO_CLS, 8, CTX_DIM), jnp.float32),
            pltpu.VMEM((BPW,), jnp.int32),
            pltpu.VMEM((BPW,), jnp.int32),
            pltpu.VMEM((BPW,), jnp.int32),
            pltpu.VMEM((N_CO_GRAPH, 4, CTX_DIM), jnp.float32),
            pltpu.VMEM((CHUNK, 4, CTX_DIM), jnp.float32),
            pltpu.SemaphoreType.DMA,
            pltpu.SemaphoreType.DMA,
            pltpu.SemaphoreType.DMA,
        ],
    )
    return run(cls_idx, cls_group_idx, graph_group_idx, ctx_single, ctx_all,
               ctx_cls, ctx_graph)
